# baseline (speedup 1.0000x reference)
import jax
import jax.numpy as jnp
from jax import lax
from jax.experimental import pallas as pl
from jax.experimental.pallas import tpu as pltpu
from jax.experimental.pallas import tpu_sc as plsc

VOCAB = 50257
D = 768
BATCH = 4
SEQ = 2048
NC = 2
NS = 16
NW = NC * NS
PPW = SEQ // NW
C = 32
HALVES = PPW // C
NCHUNK = BATCH * HALVES
LANES = 16
VECS_PER_ROW = D // LANES
NRBUF = 3


def _emb_body(ids_hbm, wte_hbm, wpe_hbm, out_hbm,
              idx_v, wpe_v, r0, r1, r2,
              g0, g1, g2, s0, s1, s2, wsem, isem):
    rows = [r0, r1, r2]
    gsems = [g0, g1, g2]
    ssems = [s0, s1, s2]

    wid = lax.axis_index("s") * NC + lax.axis_index("c")
    pos_base = wid * PPW

    def start_gather(ci):
        return pltpu.async_copy(
            wte_hbm.at[idx_v.at[ci // HALVES, pl.ds((ci % HALVES) * C, C)]],
            rows[ci % NRBUF], gsems[ci % NRBUF])

    pltpu.sync_copy(ids_hbm.at[0, pl.ds(pos_base, PPW)], idx_v.at[0])
    gdesc = [None] * NRBUF
    sdesc = [None] * NRBUF
    gdesc[0] = start_gather(0)
    wdesc = pltpu.async_copy(wpe_hbm.at[pl.ds(pos_base, PPW)], wpe_v, wsem)
    idescs = [
        pltpu.async_copy(ids_hbm.at[b, pl.ds(pos_base, PPW)], idx_v.at[b], isem)
        for b in range(1, BATCH)
    ]
    for d in idescs:
        d.wait()
    gdesc[1] = start_gather(1)

    for ci in range(NCHUNK):
        cur = ci % NRBUF
        if ci + 2 < NCHUNK:
            nb = (ci + 2) % NRBUF
            if sdesc[nb] is not None:
                sdesc[nb].wait()
            gdesc[nb] = start_gather(ci + 2)
        gdesc[cur].wait()
        if ci == 0:
            wdesc.wait()

        rbuf = rows[cur]
        h = ci % HALVES
        woff = h * C

        @plsc.parallel_loop(0, C, unroll=2)
        def add_row(r):
            for j in range(VECS_PER_ROW):
                off = j * LANES
                v = wpe_v[woff + r, pl.ds(off, LANES)]
                plsc.addupdate(rbuf.at[r, pl.ds(off, LANES)], v)

        b = ci // HALVES
        sdesc[cur] = pltpu.async_copy(
            rbuf, out_hbm.at[b, pl.ds(pos_base + woff, C)], ssems[cur])

    for k in range(NRBUF - 1):
        sdesc[(NCHUNK - 1 - k) % NRBUF].wait()


@jax.jit
def _emb(ids, wte, wpe):
    mesh = plsc.VectorSubcoreMesh(
        core_axis_name="c", subcore_axis_name="s", num_cores=NC, num_subcores=NS
    )
    return pl.kernel(
        _emb_body,
        out_type=jax.ShapeDtypeStruct((BATCH, SEQ, D), jnp.float32),
        mesh=mesh,
        scratch_types=[
            pltpu.VMEM((BATCH, PPW), jnp.int32),
            pltpu.VMEM((PPW, D), jnp.float32),
            pltpu.VMEM((C, D), jnp.float32),
            pltpu.VMEM((C, D), jnp.float32),
            pltpu.VMEM((C, D), jnp.float32),
            pltpu.SemaphoreType.DMA,
            pltpu.SemaphoreType.DMA,
            pltpu.SemaphoreType.DMA,
            pltpu.SemaphoreType.DMA,
            pltpu.SemaphoreType.DMA,
            pltpu.SemaphoreType.DMA,
            pltpu.SemaphoreType.DMA,
            pltpu.SemaphoreType.DMA,
        ],
    )(ids, wte, wpe)


def kernel(input_ids, wte, wpe):
    return _emb(input_ids, wte, wpe)
